# baseline (speedup 1.0000x reference)
import functools

import jax
import jax.numpy as jnp
from jax import lax
from jax.experimental import pallas as pl
from jax.experimental.pallas import tpu as pltpu
from jax.experimental.pallas import tpu_sc as plsc

N = 10000
NP = 10240
D = 128
H = 256
HH = 128
OUT = 128
G = 16
E = 320000
EP = 327680
CH = 64
RING = 5
NSUB = 16
CPS = EP // CH // NSUB
SLAB = NP // NSUB
TRASH = 10016

_f32 = jnp.float32


BR = 1280
NB = NP // BR
import numpy as _np
SQ = float(_np.sqrt(_np.float32(1.0 + 1e-3)))


def _dot(a, b):
    return jnp.dot(a, b, preferred_element_type=_f32)


def _pre_body(x_ref, w1, b1, w2, b2, w3, b3, wg, o_lo, o_hi):
    a = jnp.tanh((_dot(x_ref[...], w1[...]) + b1[...]) / SQ)
    a = jnp.tanh((_dot(a, w2[...]) + b2[...]) / SQ)
    a = jnp.tanh((_dot(a, w3[...]) + b3[...]) / SQ)
    h = _dot(a, wg[...])
    o_lo[...] = h[:, :HH]
    o_hi[...] = h[:, HH:]


def _bcast(i):
    return (0, 0)


def _rows(i):
    return (i, 0)


_tc_pre = pl.pallas_call(
    _pre_body,
    grid=(NB,),
    in_specs=[pl.BlockSpec((BR, D), _rows)] + [pl.BlockSpec(None, _bcast)] * 7,
    out_specs=[pl.BlockSpec((BR, HH), _rows)] * 2,
    out_shape=[jax.ShapeDtypeStruct((NP, HH), _f32)] * 2,
)


def _mid_body(g_lo, g_hi, b, w, o_lo, o_hi):
    a_lo = jnp.tanh(g_lo[...] + b[:, :HH])
    a_hi = jnp.tanh(g_hi[...] + b[:, HH:])
    h = _dot(a_lo, w[:HH, :]) + _dot(a_hi, w[HH:, :])
    o_lo[...] = h[:, :HH]
    o_hi[...] = h[:, HH:]


_tc_mid = pl.pallas_call(
    _mid_body,
    grid=(NB,),
    in_specs=[pl.BlockSpec((BR, HH), _rows)] * 2 + [pl.BlockSpec(None, _bcast)] * 2,
    out_specs=[pl.BlockSpec((BR, HH), _rows)] * 2,
    out_shape=[jax.ShapeDtypeStruct((NP, HH), _f32)] * 2,
)


def _post_body(g_lo, g_hi, b, seg, pw1, pb1, pw2, pb2, out_ref, pacc):
    i = pl.program_id(0)
    a_lo = jnp.tanh(g_lo[...] + b[:, :HH])
    a_hi = jnp.tanh(g_hi[...] + b[:, HH:])
    a = jnp.concatenate([a_lo, a_hi], axis=1)
    ids = lax.broadcasted_iota(jnp.int32, (G, BR), 0)
    onehot = (ids == seg[...]).astype(_f32)
    part = jnp.dot(onehot, a, preferred_element_type=_f32,
                   precision=lax.Precision.HIGHEST)

    @pl.when(i == 0)
    def _():
        pacc[...] = jnp.zeros_like(pacc)

    pacc[...] += part

    @pl.when(i == NB - 1)
    def _():
        p = jnp.tanh((_dot(pacc[...], pw1[...]) + pb1[...]) / SQ)
        out_ref[...] = (_dot(p, pw2[...]) + pb2[...]) / SQ


_tc_post = pl.pallas_call(
    _post_body,
    grid=(NB,),
    in_specs=([pl.BlockSpec((BR, HH), _rows)] * 2
              + [pl.BlockSpec(None, _bcast),
                 pl.BlockSpec((1, BR), lambda i: (0, i))]
              + [pl.BlockSpec(None, _bcast)] * 4),
    out_specs=pl.BlockSpec(None, _bcast),
    out_shape=jax.ShapeDtypeStruct((G, OUT), _f32),
    scratch_shapes=[pltpu.VMEM((G, H), _f32)],
)


_sc_mesh = plsc.VectorSubcoreMesh(core_axis_name="c", subcore_axis_name="s")


@functools.partial(
    pl.kernel,
    mesh=_sc_mesh,
    out_type=[jax.ShapeDtypeStruct((NP, HH), _f32)] * 2,
    scratch_types=[
        pltpu.VMEM((CH,), jnp.int32),
        pltpu.VMEM((CH,), jnp.int32),
        pltpu.VMEM((CH,), jnp.int32),
        pltpu.VMEM((CH,), jnp.int32),
        pltpu.VMEM((CH, HH), _f32),
        pltpu.VMEM((CH, HH), _f32),
        pltpu.VMEM_SHARED((NP, HH), _f32),
        pltpu.SemaphoreType.DMA,
        pltpu.SemaphoreType.DMA,
        pltpu.SemaphoreType.DMA,
        pltpu.SemaphoreType.DMA,
    ],
)
def _sc_agg(h_lo, h_hi, src2d, dst2d, agg_lo, agg_hi,
            s0, d0, s1, d1, rows0, rows1, acc,
            semi0, semi1, semg0, semg1):
    c = lax.axis_index("c")
    s = lax.axis_index("s")
    base = s * CPS

    def start_idx(sbuf, dbuf, k, semi):
        kc = jnp.minimum(k, CPS - 1) + base
        pltpu.make_async_copy(src2d.at[kc], sbuf, semi).start()
        pltpu.make_async_copy(dst2d.at[kc], dbuf, semi).start()

    def wait_idx(sbuf, dbuf, semi):
        pltpu.make_async_copy(src2d.at[base], sbuf, semi).wait()
        pltpu.make_async_copy(dst2d.at[base], dbuf, semi).wait()

    def run(href, aggref):
        def zbody(i, _):
            rr = i // (HH // 16)
            col = (i % (HH // 16)) * 16
            rows0[rr, pl.ds(col, 16)] = jnp.zeros((16,), _f32)
            return 0
        lax.fori_loop(0, CH * (HH // 16), zbody, 0)
        for j in range(SLAB // CH):
            pltpu.sync_copy(rows0, acc.at[pl.ds(s * SLAB + j * CH, CH)])
        plsc.subcore_barrier()

        start_idx(s0, d0, 0, semi0)
        start_idx(s1, d1, 1, semi1)
        wait_idx(s0, d0, semi0)
        pltpu.make_async_copy(href.at[s0], rows0, semg0).start()

        def body(kk, _):
            k0 = 2 * kk
            pltpu.make_async_copy(href.at[s0], rows0, semg0).wait()
            wait_idx(s1, d1, semi1)
            pltpu.make_async_copy(href.at[s1], rows1, semg1).start()
            pltpu.sync_copy(rows0, acc.at[d0], add=True)
            start_idx(s0, d0, k0 + 2, semi0)
            pltpu.make_async_copy(href.at[s1], rows1, semg1).wait()
            pltpu.sync_copy(rows1, acc.at[d1], add=True)
            start_idx(s1, d1, k0 + 3, semi1)
            wait_idx(s0, d0, semi0)
            pltpu.make_async_copy(href.at[s0], rows0, semg0).start()
            return 0
        lax.fori_loop(0, CPS // 2, body, 0)
        pltpu.make_async_copy(href.at[s0], rows0, semg0).wait()
        wait_idx(s1, d1, semi1)

        plsc.subcore_barrier()
        pltpu.sync_copy(acc.at[pl.ds(s * SLAB, SLAB)],
                        aggref.at[pl.ds(s * SLAB, SLAB)])

    pl.when(c == 0)(lambda: run(h_lo, agg_lo))
    pl.when(c == 1)(lambda: run(h_hi, agg_hi))


def kernel(x, edge_index, segment_ids, pre_Ws, pre_bs, gcn_Ws, gcn_bs,
           post_Ws, post_bs):
    x_pad = jnp.pad(x, ((0, NP - N), (0, 0)))
    seg = jnp.pad(segment_ids.astype(jnp.int32), (0, NP - N),
                  constant_values=G).reshape(1, NP)
    src2d = jnp.pad(edge_index[0], (0, EP - E)).reshape(EP // CH, CH)
    dst2d = jnp.pad(edge_index[1], (0, EP - E),
                    constant_values=TRASH).reshape(EP // CH, CH)

    pbs = [b.reshape(1, -1) for b in pre_bs]
    qbs = [b.reshape(1, -1) for b in post_bs]
    gbs = [b.reshape(1, -1) for b in gcn_bs]

    h_lo, h_hi = _tc_pre(x_pad, pre_Ws[0], pbs[0], pre_Ws[1], pbs[1],
                         pre_Ws[2], pbs[2], gcn_Ws[0])
    for i in range(5):
        agg_lo, agg_hi = _sc_agg(h_lo, h_hi, src2d, dst2d)
        if i < 4:
            h_lo, h_hi = _tc_mid(agg_lo, agg_hi, gbs[i], gcn_Ws[i + 1])
    return _tc_post(agg_lo, agg_hi, gbs[4], seg, post_Ws[0], qbs[0],
                    post_Ws[1], qbs[1])


# speedup vs baseline: 1.1438x; 1.1438x over previous
import functools

import jax
import jax.numpy as jnp
from jax import lax
from jax.experimental import pallas as pl
from jax.experimental.pallas import tpu as pltpu
from jax.experimental.pallas import tpu_sc as plsc

N = 10000
NP = 10240
D = 128
H = 256
HH = 128
OUT = 128
G = 16
E = 320000
EP = 327680
CH = 128
NSUB = 16
CPS = EP // CH // NSUB
SLAB = NP // NSUB
TRASH = 10016

_f32 = jnp.float32


BR = 1280
NB = NP // BR
import numpy as _np
SQ = float(_np.sqrt(_np.float32(1.0 + 1e-3)))


def _dot(a, b):
    return jnp.dot(a, b, preferred_element_type=_f32)


def _pre_body(x_ref, w1, b1, w2, b2, w3, b3, wg, o_lo, o_hi):
    a = jnp.tanh((_dot(x_ref[...], w1[...]) + b1[...]) / SQ)
    a = jnp.tanh((_dot(a, w2[...]) + b2[...]) / SQ)
    a = jnp.tanh((_dot(a, w3[...]) + b3[...]) / SQ)
    h = _dot(a, wg[...])
    o_lo[...] = h[:, :HH]
    o_hi[...] = h[:, HH:]


def _bcast(i):
    return (0, 0)


def _rows(i):
    return (i, 0)


_tc_pre = pl.pallas_call(
    _pre_body,
    grid=(NB,),
    in_specs=[pl.BlockSpec((BR, D), _rows)] + [pl.BlockSpec(None, _bcast)] * 7,
    out_specs=[pl.BlockSpec((BR, HH), _rows)] * 2,
    out_shape=[jax.ShapeDtypeStruct((NP, HH), _f32)] * 2,
)


def _mid_body(g_lo, g_hi, b, w, o_lo, o_hi):
    a_lo = jnp.tanh(g_lo[...] + b[:, :HH])
    a_hi = jnp.tanh(g_hi[...] + b[:, HH:])
    h = _dot(a_lo, w[:HH, :]) + _dot(a_hi, w[HH:, :])
    o_lo[...] = h[:, :HH]
    o_hi[...] = h[:, HH:]


_tc_mid = pl.pallas_call(
    _mid_body,
    grid=(NB,),
    in_specs=[pl.BlockSpec((BR, HH), _rows)] * 2 + [pl.BlockSpec(None, _bcast)] * 2,
    out_specs=[pl.BlockSpec((BR, HH), _rows)] * 2,
    out_shape=[jax.ShapeDtypeStruct((NP, HH), _f32)] * 2,
)


def _post_body(g_lo, g_hi, b, seg, pw1, pb1, pw2, pb2, out_ref, pacc):
    i = pl.program_id(0)
    a_lo = jnp.tanh(g_lo[...] + b[:, :HH])
    a_hi = jnp.tanh(g_hi[...] + b[:, HH:])
    a = jnp.concatenate([a_lo, a_hi], axis=1)
    ids = lax.broadcasted_iota(jnp.int32, (G, BR), 0)
    onehot = (ids == seg[...]).astype(_f32)
    part = jnp.dot(onehot, a, preferred_element_type=_f32,
                   precision=lax.Precision.HIGHEST)

    @pl.when(i == 0)
    def _():
        pacc[...] = jnp.zeros_like(pacc)

    pacc[...] += part

    @pl.when(i == NB - 1)
    def _():
        p = jnp.tanh((_dot(pacc[...], pw1[...]) + pb1[...]) / SQ)
        out_ref[...] = (_dot(p, pw2[...]) + pb2[...]) / SQ


_tc_post = pl.pallas_call(
    _post_body,
    grid=(NB,),
    in_specs=([pl.BlockSpec((BR, HH), _rows)] * 2
              + [pl.BlockSpec(None, _bcast),
                 pl.BlockSpec((1, BR), lambda i: (0, i))]
              + [pl.BlockSpec(None, _bcast)] * 4),
    out_specs=pl.BlockSpec(None, _bcast),
    out_shape=jax.ShapeDtypeStruct((G, OUT), _f32),
    scratch_shapes=[pltpu.VMEM((G, H), _f32)],
)


_sc_mesh = plsc.VectorSubcoreMesh(core_axis_name="c", subcore_axis_name="s")


@functools.partial(
    pl.kernel,
    mesh=_sc_mesh,
    out_type=[jax.ShapeDtypeStruct((NP, HH), _f32)] * 2,
    scratch_types=[
        pltpu.VMEM((CH,), jnp.int32),
        pltpu.VMEM((CH,), jnp.int32),
        pltpu.VMEM((CH,), jnp.int32),
        pltpu.VMEM((CH,), jnp.int32),
        pltpu.VMEM((CH, HH), _f32),
        pltpu.VMEM((CH, HH), _f32),
        pltpu.VMEM_SHARED((NP, HH), _f32),
        pltpu.SemaphoreType.DMA,
        pltpu.SemaphoreType.DMA,
        pltpu.SemaphoreType.DMA,
        pltpu.SemaphoreType.DMA,
    ],
)
def _sc_agg(h_lo, h_hi, src2d, dst2d, agg_lo, agg_hi,
            s0, d0, s1, d1, rows0, rows1, acc,
            semi0, semi1, semg0, semg1):
    c = lax.axis_index("c")
    s = lax.axis_index("s")
    base = s * CPS

    def start_idx(sbuf, dbuf, k, semi):
        kc = jnp.minimum(k, CPS - 1) + base
        pltpu.make_async_copy(src2d.at[kc], sbuf, semi).start()
        pltpu.make_async_copy(dst2d.at[kc], dbuf, semi).start()

    def wait_idx(sbuf, dbuf, semi):
        pltpu.make_async_copy(src2d.at[base], sbuf, semi).wait()
        pltpu.make_async_copy(dst2d.at[base], dbuf, semi).wait()

    def run(href, aggref):
        def zbody(i, _):
            rr = i // (HH // 16)
            col = (i % (HH // 16)) * 16
            rows0[rr, pl.ds(col, 16)] = jnp.zeros((16,), _f32)
            return 0
        lax.fori_loop(0, CH * (HH // 16), zbody, 0)
        for j in range(SLAB // CH):
            pltpu.sync_copy(rows0, acc.at[pl.ds(s * SLAB + j * CH, CH)])
        plsc.subcore_barrier()

        start_idx(s0, d0, 0, semi0)
        start_idx(s1, d1, 1, semi1)
        wait_idx(s0, d0, semi0)
        pltpu.make_async_copy(href.at[s0], rows0, semg0).start()

        def body(kk, _):
            k0 = 2 * kk
            pltpu.make_async_copy(href.at[s0], rows0, semg0).wait()
            wait_idx(s1, d1, semi1)
            pltpu.make_async_copy(href.at[s1], rows1, semg1).start()
            pltpu.sync_copy(rows0, acc.at[d0], add=True)
            start_idx(s0, d0, k0 + 2, semi0)
            pltpu.make_async_copy(href.at[s1], rows1, semg1).wait()
            pltpu.sync_copy(rows1, acc.at[d1], add=True)
            start_idx(s1, d1, k0 + 3, semi1)
            wait_idx(s0, d0, semi0)
            pltpu.make_async_copy(href.at[s0], rows0, semg0).start()
            return 0
        lax.fori_loop(0, CPS // 2, body, 0)
        pltpu.make_async_copy(href.at[s0], rows0, semg0).wait()
        wait_idx(s1, d1, semi1)

        plsc.subcore_barrier()
        pltpu.sync_copy(acc.at[pl.ds(s * SLAB, SLAB)],
                        aggref.at[pl.ds(s * SLAB, SLAB)])

    pl.when(c == 0)(lambda: run(h_lo, agg_lo))
    pl.when(c == 1)(lambda: run(h_hi, agg_hi))


def kernel(x, edge_index, segment_ids, pre_Ws, pre_bs, gcn_Ws, gcn_bs,
           post_Ws, post_bs):
    x_pad = jnp.pad(x, ((0, NP - N), (0, 0)))
    seg = jnp.pad(segment_ids.astype(jnp.int32), (0, NP - N),
                  constant_values=G).reshape(1, NP)
    src2d = jnp.pad(edge_index[0], (0, EP - E)).reshape(EP // CH, CH)
    dst2d = jnp.pad(edge_index[1], (0, EP - E),
                    constant_values=TRASH).reshape(EP // CH, CH)

    pbs = [b.reshape(1, -1) for b in pre_bs]
    qbs = [b.reshape(1, -1) for b in post_bs]
    gbs = [b.reshape(1, -1) for b in gcn_bs]

    h_lo, h_hi = _tc_pre(x_pad, pre_Ws[0], pbs[0], pre_Ws[1], pbs[1],
                         pre_Ws[2], pbs[2], gcn_Ws[0])
    for i in range(5):
        agg_lo, agg_hi = _sc_agg(h_lo, h_hi, src2d, dst2d)
        if i < 4:
            h_lo, h_hi = _tc_mid(agg_lo, agg_hi, gbs[i], gcn_Ws[i + 1])
    return _tc_post(agg_lo, agg_hi, gbs[4], seg, post_Ws[0], qbs[0],
                    post_Ws[1], qbs[1])


# speedup vs baseline: 3.6204x; 3.1651x over previous
import functools

import jax
import jax.numpy as jnp
from jax import lax
from jax.experimental import pallas as pl
from jax.experimental.pallas import tpu as pltpu
from jax.experimental.pallas import tpu_sc as plsc

N = 10000
NP = 10240
D = 128
H = 256
HH = 128
OUT = 128
G = 16
E = 320000
EP = 327680
CH = 128
NSUB = 16
CPS = EP // CH // NSUB
SLAB = NP // NSUB

_f32 = jnp.float32


BR = 1280
NB = NP // BR
import numpy as _np
SQ = float(_np.sqrt(_np.float32(1.0 + 1e-3)))


def _dot(a, b):
    return jnp.dot(a, b, preferred_element_type=_f32)


def _pre_body(x_ref, w1, b1, w2, b2, w3, b3, wg, o_lo, o_hi):
    a = jnp.tanh((_dot(x_ref[...], w1[...]) + b1[...]) / SQ)
    a = jnp.tanh((_dot(a, w2[...]) + b2[...]) / SQ)
    a = jnp.tanh((_dot(a, w3[...]) + b3[...]) / SQ)
    h = _dot(a, wg[...])
    o_lo[...] = h[:, :HH]
    o_hi[...] = h[:, HH:]


def _bcast(i):
    return (0, 0)


def _rows(i):
    return (i, 0)


_tc_pre = pl.pallas_call(
    _pre_body,
    grid=(NB,),
    in_specs=[pl.BlockSpec((BR, D), _rows)] + [pl.BlockSpec(None, _bcast)] * 7,
    out_specs=[pl.BlockSpec((BR, HH), _rows)] * 2,
    out_shape=[jax.ShapeDtypeStruct((NP, HH), _f32)] * 2,
)


def _mid_body(g_lo, g_hi, b, w, o_lo, o_hi):
    a_lo = jnp.tanh(g_lo[...] + b[:, :HH])
    a_hi = jnp.tanh(g_hi[...] + b[:, HH:])
    h = _dot(a_lo, w[:HH, :]) + _dot(a_hi, w[HH:, :])
    o_lo[...] = h[:, :HH]
    o_hi[...] = h[:, HH:]


_tc_mid = pl.pallas_call(
    _mid_body,
    grid=(NB,),
    in_specs=[pl.BlockSpec((BR, HH), _rows)] * 2 + [pl.BlockSpec(None, _bcast)] * 2,
    out_specs=[pl.BlockSpec((BR, HH), _rows)] * 2,
    out_shape=[jax.ShapeDtypeStruct((NP, HH), _f32)] * 2,
)


def _post_body(g_lo, g_hi, b, seg, pw1, pb1, pw2, pb2, out_ref, pacc):
    i = pl.program_id(0)
    a_lo = jnp.tanh(g_lo[...] + b[:, :HH])
    a_hi = jnp.tanh(g_hi[...] + b[:, HH:])
    a = jnp.concatenate([a_lo, a_hi], axis=1)
    ids = lax.broadcasted_iota(jnp.int32, (G, BR), 0)
    onehot = (ids == seg[...]).astype(_f32)
    part = jnp.dot(onehot, a, preferred_element_type=_f32,
                   precision=lax.Precision.HIGHEST)

    @pl.when(i == 0)
    def _():
        pacc[...] = jnp.zeros_like(pacc)

    pacc[...] += part

    @pl.when(i == NB - 1)
    def _():
        p = jnp.tanh((_dot(pacc[...], pw1[...]) + pb1[...]) / SQ)
        out_ref[...] = (_dot(p, pw2[...]) + pb2[...]) / SQ


_tc_post = pl.pallas_call(
    _post_body,
    grid=(NB,),
    in_specs=([pl.BlockSpec((BR, HH), _rows)] * 2
              + [pl.BlockSpec(None, _bcast),
                 pl.BlockSpec((1, BR), lambda i: (0, i))]
              + [pl.BlockSpec(None, _bcast)] * 4),
    out_specs=pl.BlockSpec(None, _bcast),
    out_shape=jax.ShapeDtypeStruct((G, OUT), _f32),
    scratch_shapes=[pltpu.VMEM((G, H), _f32)],
)


_sc_mesh = plsc.VectorSubcoreMesh(core_axis_name="c", subcore_axis_name="s")


@functools.partial(
    pl.kernel,
    mesh=_sc_mesh,
    out_type=[jax.ShapeDtypeStruct((NP, HH), _f32)] * 2,
    scratch_types=(
        [pltpu.VMEM((CH,), jnp.int32)] * 8
        + [pltpu.VMEM((CH, HH), _f32)] * 2
        + [pltpu.VMEM_SHARED((NP, HH), _f32)]
        + [pltpu.SemaphoreType.DMA] * 6
    ),
)
def _sc_agg(h_lo, h_hi, src2d, dst2d, agg_lo, agg_hi,
            s0, d0, s1, d1, s2, d2, s3, d3, rows0, rows1, acc,
            semi0, semi1, semi2, semi3, semg0, semg1):
    c = lax.axis_index("c")
    s = lax.axis_index("s")
    base = s * CPS
    sbufs = [s0, s1, s2, s3]
    dbufs = [d0, d1, d2, d3]
    isems = [semi0, semi1, semi2, semi3]
    rows = [rows0, rows1]
    gsems = [semg0, semg1]

    def start_idx(p, k):
        kc = jnp.minimum(k, CPS - 1) + base
        pltpu.make_async_copy(src2d.at[kc], sbufs[p], isems[p]).start()
        pltpu.make_async_copy(dst2d.at[kc], dbufs[p], isems[p]).start()

    def wait_idx(p):
        pltpu.make_async_copy(src2d.at[base], sbufs[p], isems[p]).wait()
        pltpu.make_async_copy(dst2d.at[base], dbufs[p], isems[p]).wait()

    def run(href, aggref):
        def zbody(i, _):
            rr = i // (HH // 16)
            col = (i % (HH // 16)) * 16
            rows0[rr, pl.ds(col, 16)] = jnp.zeros((16,), _f32)
            return 0
        lax.fori_loop(0, CH * (HH // 16), zbody, 0)
        for j in range(SLAB // CH):
            pltpu.sync_copy(rows0, acc.at[pl.ds(s * SLAB + j * CH, CH)])
        plsc.subcore_barrier()

        for p in range(4):
            start_idx(p, p)
        wait_idx(0)
        pltpu.make_async_copy(href.at[s0], rows0, gsems[0]).start()
        wait_idx(1)
        pltpu.make_async_copy(href.at[s1], rows1, gsems[1]).start()

        def body(t, _):
            k0 = 4 * t
            for j in range(4):
                r, p, pn = j % 2, j % 4, (j + 2) % 4
                pltpu.make_async_copy(href.at[sbufs[p]], rows[r],
                                      gsems[r]).wait()
                pltpu.sync_copy(rows[r], acc.at[dbufs[p]], add=True)
                start_idx(p, k0 + j + 4)
                wait_idx(pn)
                pltpu.make_async_copy(href.at[sbufs[pn]], rows[r],
                                      gsems[r]).start()
            return 0
        lax.fori_loop(0, CPS // 4, body, 0)
        pltpu.make_async_copy(href.at[s0], rows0, gsems[0]).wait()
        pltpu.make_async_copy(href.at[s1], rows1, gsems[1]).wait()
        wait_idx(2)
        wait_idx(3)

        plsc.subcore_barrier()
        pltpu.sync_copy(acc.at[pl.ds(s * SLAB, SLAB)],
                        aggref.at[pl.ds(s * SLAB, SLAB)])

    pl.when(c == 0)(lambda: run(h_lo, agg_lo))
    pl.when(c == 1)(lambda: run(h_hi, agg_hi))


def kernel(x, edge_index, segment_ids, pre_Ws, pre_bs, gcn_Ws, gcn_bs,
           post_Ws, post_bs):
    x_pad = jnp.pad(x, ((0, NP - N), (0, 0)))
    seg = jnp.pad(segment_ids.astype(jnp.int32), (0, NP - N),
                  constant_values=G).reshape(1, NP)
    pad_ar = jnp.arange(EP - E, dtype=jnp.int32)
    src2d = jnp.concatenate(
        [edge_index[0], pad_ar % N]).reshape(EP // CH, CH)
    dst2d = jnp.concatenate(
        [edge_index[1], N + pad_ar % (NP - N)]).reshape(EP // CH, CH)

    pbs = [b.reshape(1, -1) for b in pre_bs]
    qbs = [b.reshape(1, -1) for b in post_bs]
    gbs = [b.reshape(1, -1) for b in gcn_bs]

    h_lo, h_hi = _tc_pre(x_pad, pre_Ws[0], pbs[0], pre_Ws[1], pbs[1],
                         pre_Ws[2], pbs[2], gcn_Ws[0])
    for i in range(5):
        agg_lo, agg_hi = _sc_agg(h_lo, h_hi, src2d, dst2d)
        if i < 4:
            h_lo, h_hi = _tc_mid(agg_lo, agg_hi, gbs[i], gcn_Ws[i + 1])
    return _tc_post(agg_lo, agg_hi, gbs[4], seg, post_Ws[0], qbs[0],
                    post_Ws[1], qbs[1])


# speedup vs baseline: 3.6628x; 1.0117x over previous
import functools

import jax
import jax.numpy as jnp
from jax import lax
from jax.experimental import pallas as pl
from jax.experimental.pallas import tpu as pltpu
from jax.experimental.pallas import tpu_sc as plsc

N = 10000
NP = 10240
D = 128
H = 256
HH = 128
OUT = 128
G = 16
E = 320000
CH = 120
NSUB = 16
CPS = 168
EP = CH * NSUB * CPS
SLAB = NP // NSUB

_f32 = jnp.float32


BR = 1280
NB = NP // BR
import numpy as _np
SQ = float(_np.sqrt(_np.float32(1.0 + 1e-3)))


def _dot(a, b):
    return jnp.dot(a, b, preferred_element_type=_f32)


def _pre_body(x_ref, w1, b1, w2, b2, w3, b3, wg, o_lo, o_hi):
    a = jnp.tanh((_dot(x_ref[...], w1[...]) + b1[...]) / SQ)
    a = jnp.tanh((_dot(a, w2[...]) + b2[...]) / SQ)
    a = jnp.tanh((_dot(a, w3[...]) + b3[...]) / SQ)
    h = _dot(a, wg[...])
    o_lo[...] = h[:, :HH]
    o_hi[...] = h[:, HH:]


def _bcast(i):
    return (0, 0)


def _rows(i):
    return (i, 0)


_tc_pre = pl.pallas_call(
    _pre_body,
    grid=(NB,),
    in_specs=[pl.BlockSpec((BR, D), _rows)] + [pl.BlockSpec(None, _bcast)] * 7,
    out_specs=[pl.BlockSpec((BR, HH), _rows)] * 2,
    out_shape=[jax.ShapeDtypeStruct((NP, HH), _f32)] * 2,
)


def _mid_body(g_lo, g_hi, b, w, o_lo, o_hi):
    a_lo = jnp.tanh(g_lo[...] + b[:, :HH])
    a_hi = jnp.tanh(g_hi[...] + b[:, HH:])
    h = _dot(a_lo, w[:HH, :]) + _dot(a_hi, w[HH:, :])
    o_lo[...] = h[:, :HH]
    o_hi[...] = h[:, HH:]


_tc_mid = pl.pallas_call(
    _mid_body,
    grid=(NB,),
    in_specs=[pl.BlockSpec((BR, HH), _rows)] * 2 + [pl.BlockSpec(None, _bcast)] * 2,
    out_specs=[pl.BlockSpec((BR, HH), _rows)] * 2,
    out_shape=[jax.ShapeDtypeStruct((NP, HH), _f32)] * 2,
)


def _post_body(g_lo, g_hi, b, seg, pw1, pb1, pw2, pb2, out_ref, pacc):
    i = pl.program_id(0)
    a_lo = jnp.tanh(g_lo[...] + b[:, :HH])
    a_hi = jnp.tanh(g_hi[...] + b[:, HH:])
    a = jnp.concatenate([a_lo, a_hi], axis=1)
    ids = lax.broadcasted_iota(jnp.int32, (G, BR), 0)
    onehot = (ids == seg[...]).astype(_f32)
    part = jnp.dot(onehot, a, preferred_element_type=_f32,
                   precision=lax.Precision.HIGHEST)

    @pl.when(i == 0)
    def _():
        pacc[...] = jnp.zeros_like(pacc)

    pacc[...] += part

    @pl.when(i == NB - 1)
    def _():
        p = jnp.tanh((_dot(pacc[...], pw1[...]) + pb1[...]) / SQ)
        out_ref[...] = (_dot(p, pw2[...]) + pb2[...]) / SQ


_tc_post = pl.pallas_call(
    _post_body,
    grid=(NB,),
    in_specs=([pl.BlockSpec((BR, HH), _rows)] * 2
              + [pl.BlockSpec(None, _bcast),
                 pl.BlockSpec((1, BR), lambda i: (0, i))]
              + [pl.BlockSpec(None, _bcast)] * 4),
    out_specs=pl.BlockSpec(None, _bcast),
    out_shape=jax.ShapeDtypeStruct((G, OUT), _f32),
    scratch_shapes=[pltpu.VMEM((G, H), _f32)],
)


_sc_mesh = plsc.VectorSubcoreMesh(core_axis_name="c", subcore_axis_name="s")


@functools.partial(
    pl.kernel,
    mesh=_sc_mesh,
    out_type=[jax.ShapeDtypeStruct((NP, HH), _f32)] * 2,
    scratch_types=(
        [pltpu.VMEM((CH,), jnp.int32)] * 6
        + [pltpu.VMEM((CH, HH), _f32)] * 3
        + [pltpu.VMEM_SHARED((NP, HH), _f32)]
        + [pltpu.SemaphoreType.DMA] * 6
    ),
)
def _sc_agg(h_lo, h_hi, src2d, dst2d, agg_lo, agg_hi,
            s0, d0, s1, d1, s2, d2, rows0, rows1, rows2, acc,
            semi0, semi1, semi2, semg0, semg1, semg2):
    c = lax.axis_index("c")
    s = lax.axis_index("s")
    base = s * CPS
    sbufs = [s0, s1, s2]
    dbufs = [d0, d1, d2]
    isems = [semi0, semi1, semi2]
    rows = [rows0, rows1, rows2]
    gsems = [semg0, semg1, semg2]

    def start_idx(p, k):
        kc = jnp.minimum(k, CPS - 1) + base
        pltpu.make_async_copy(src2d.at[kc], sbufs[p], isems[p]).start()
        pltpu.make_async_copy(dst2d.at[kc], dbufs[p], isems[p]).start()

    def wait_idx(p):
        pltpu.make_async_copy(src2d.at[base], sbufs[p], isems[p]).wait()
        pltpu.make_async_copy(dst2d.at[base], dbufs[p], isems[p]).wait()

    def run(href, aggref):
        def zbody(i, _):
            rr = i // (HH // 16)
            col = (i % (HH // 16)) * 16
            rows0[rr, pl.ds(col, 16)] = jnp.zeros((16,), _f32)
            return 0
        lax.fori_loop(0, CH * (HH // 16), zbody, 0)
        off = 0
        while off < SLAB:
            n = min(CH, SLAB - off)
            pltpu.sync_copy(rows0.at[pl.ds(0, n)],
                            acc.at[pl.ds(s * SLAB + off, n)])
            off += n
        plsc.subcore_barrier()

        for p in range(3):
            start_idx(p, p)
        wait_idx(0)
        pltpu.make_async_copy(href.at[s0], rows0, gsems[0]).start()
        wait_idx(1)
        pltpu.make_async_copy(href.at[s1], rows1, gsems[1]).start()

        def body(t, _):
            k0 = 3 * t
            for j in range(3):
                p, pn = j % 3, (j + 2) % 3
                pltpu.make_async_copy(href.at[sbufs[p]], rows[p],
                                      gsems[p]).wait()
                pltpu.sync_copy(rows[p], acc.at[dbufs[p]], add=True)
                start_idx(p, k0 + j + 3)
                wait_idx(pn)
                pltpu.make_async_copy(href.at[sbufs[pn]], rows[pn],
                                      gsems[pn]).start()
            return 0
        lax.fori_loop(0, CPS // 3, body, 0)
        pltpu.make_async_copy(href.at[s0], rows0, gsems[0]).wait()
        pltpu.make_async_copy(href.at[s1], rows1, gsems[1]).wait()
        wait_idx(2)

        plsc.subcore_barrier()
        pltpu.sync_copy(acc.at[pl.ds(s * SLAB, SLAB)],
                        aggref.at[pl.ds(s * SLAB, SLAB)])

    pl.when(c == 0)(lambda: run(h_lo, agg_lo))
    pl.when(c == 1)(lambda: run(h_hi, agg_hi))


def kernel(x, edge_index, segment_ids, pre_Ws, pre_bs, gcn_Ws, gcn_bs,
           post_Ws, post_bs):
    x_pad = jnp.pad(x, ((0, NP - N), (0, 0)))
    seg = jnp.pad(segment_ids.astype(jnp.int32), (0, NP - N),
                  constant_values=G).reshape(1, NP)
    pad_ar = jnp.arange(EP - E, dtype=jnp.int32)
    src2d = jnp.concatenate(
        [edge_index[0], pad_ar % N]).reshape(EP // CH, CH)
    dst2d = jnp.concatenate(
        [edge_index[1], N + pad_ar % (NP - N)]).reshape(EP // CH, CH)

    pbs = [b.reshape(1, -1) for b in pre_bs]
    qbs = [b.reshape(1, -1) for b in post_bs]
    gbs = [b.reshape(1, -1) for b in gcn_bs]

    h_lo, h_hi = _tc_pre(x_pad, pre_Ws[0], pbs[0], pre_Ws[1], pbs[1],
                         pre_Ws[2], pbs[2], gcn_Ws[0])
    for i in range(5):
        agg_lo, agg_hi = _sc_agg(h_lo, h_hi, src2d, dst2d)
        if i < 4:
            h_lo, h_hi = _tc_mid(agg_lo, agg_hi, gbs[i], gcn_Ws[i + 1])
    return _tc_post(agg_lo, agg_hi, gbs[4], seg, post_Ws[0], qbs[0],
                    post_Ws[1], qbs[1])
